# baseline (speedup 1.0000x reference)
import functools

import jax
import jax.numpy as jnp
from jax.experimental import pallas as pl
from jax.experimental.pallas import tpu as pltpu

_VOCAB = 1000
_BB = 16
_TPAD = 56


def _fused_kernel(idx_ref, tgt_ref, table_ref, out_ref, loss_ref, aug_ref):
    @pl.when(pl.program_id(0) == 0)
    def _prep():
        tab = table_ref[...]
        m = jnp.max(tab, axis=1, keepdims=True)
        lse = m + jnp.log(jnp.sum(jnp.exp(tab - m), axis=1, keepdims=True))
        hi = lse.astype(jnp.bfloat16)
        lo = (lse - hi.astype(jnp.float32)).astype(jnp.bfloat16)
        aug_ref[:, :_VOCAB] = tab.astype(jnp.bfloat16)
        aug_ref[:, _VOCAB:_VOCAB + 1] = hi
        aug_ref[:, _VOCAB + 1:_VOCAB + 2] = lo
        loss_ref[...] = jnp.zeros((1, 1), jnp.float32)

    rows = _BB * _TPAD
    idx = idx_ref[0, 0, :]
    tgt = tgt_ref[0, 0, :]
    iota = jax.lax.broadcasted_iota(jnp.int32, (rows, _VOCAB), 1)
    onehot = (idx[:, None] == iota).astype(jnp.bfloat16)
    res = jnp.dot(onehot, aug_ref[...], preferred_element_type=jnp.float32)
    res3 = res.reshape(_BB, _TPAD, _VOCAB + 2)
    out_ref[...] = res3[:, :out_ref.shape[1], :_VOCAB]
    lse_sum = jnp.sum(res[:, _VOCAB:_VOCAB + 2])
    picked = jnp.sum(jnp.where(tgt[:, None] == iota, res[:, :_VOCAB], 0.0))
    loss_ref[...] += (lse_sum - picked).reshape(1, 1)


@functools.partial(jax.jit, static_argnames=())
def kernel(table, index, targets):
    b, t = index.shape
    n = b * t
    nblk = b // _BB
    rows = _BB * _TPAD
    pad = ((0, 0), (0, _TPAD - t))
    idx = jnp.pad(index.astype(jnp.int32), pad, constant_values=_VOCAB)
    tgt = jnp.pad(targets.astype(jnp.int32), pad, constant_values=_VOCAB)
    idx = idx.reshape(nblk, 1, rows)
    tgt = tgt.reshape(nblk, 1, rows)

    logits, loss_sum = pl.pallas_call(
        _fused_kernel,
        grid=(nblk,),
        in_specs=[
            pl.BlockSpec((1, 1, rows), lambda i: (i, 0, 0)),
            pl.BlockSpec((1, 1, rows), lambda i: (i, 0, 0)),
            pl.BlockSpec((_VOCAB, _VOCAB), lambda i: (0, 0)),
        ],
        out_specs=[
            pl.BlockSpec((_BB, t, _VOCAB), lambda i: (i, 0, 0)),
            pl.BlockSpec((1, 1), lambda i: (0, 0)),
        ],
        out_shape=[
            jax.ShapeDtypeStruct((b, t, _VOCAB), jnp.float32),
            jax.ShapeDtypeStruct((1, 1), jnp.float32),
        ],
        scratch_shapes=[pltpu.VMEM((_VOCAB, _VOCAB + 2), jnp.bfloat16)],
    )(idx, tgt, table)

    loss = loss_sum[0, 0] / n
    return (logits, loss)


# speedup vs baseline: 1.0199x; 1.0199x over previous
import functools

import jax
import jax.numpy as jnp
from jax.experimental import pallas as pl
from jax.experimental.pallas import tpu as pltpu

_VOCAB = 1000
_BB = 32
_TPAD = 56


def _fused_kernel(idx_ref, tgt_ref, table_ref, out_ref, loss_ref, aug_ref):
    @pl.when(pl.program_id(0) == 0)
    def _prep():
        tab = table_ref[...]
        m = jnp.max(tab, axis=1, keepdims=True)
        lse = m + jnp.log(jnp.sum(jnp.exp(tab - m), axis=1, keepdims=True))
        hi = lse.astype(jnp.bfloat16)
        lo = (lse - hi.astype(jnp.float32)).astype(jnp.bfloat16)
        aug_ref[:, :_VOCAB] = tab.astype(jnp.bfloat16)
        aug_ref[:, _VOCAB:_VOCAB + 1] = hi
        aug_ref[:, _VOCAB + 1:_VOCAB + 2] = lo
        loss_ref[...] = jnp.zeros((1, 1), jnp.float32)

    rows = _BB * _TPAD
    idx = idx_ref[0, 0, :]
    tgt = tgt_ref[0, 0, :]
    iota = jax.lax.broadcasted_iota(jnp.int32, (rows, _VOCAB), 1)
    onehot = (idx[:, None] == iota).astype(jnp.bfloat16)
    res = jnp.dot(onehot, aug_ref[...], preferred_element_type=jnp.float32)
    res3 = res.reshape(_BB, _TPAD, _VOCAB + 2)
    out_ref[...] = res3[:, :out_ref.shape[1], :_VOCAB]
    lse_sum = jnp.sum(res[:, _VOCAB:_VOCAB + 2])
    picked = jnp.sum(jnp.where(tgt[:, None] == iota, res[:, :_VOCAB], 0.0))
    loss_ref[...] += (lse_sum - picked).reshape(1, 1)


@functools.partial(jax.jit, static_argnames=())
def kernel(table, index, targets):
    b, t = index.shape
    n = b * t
    nblk = b // _BB
    rows = _BB * _TPAD
    pad = ((0, 0), (0, _TPAD - t))
    idx = jnp.pad(index.astype(jnp.int32), pad, constant_values=_VOCAB)
    tgt = jnp.pad(targets.astype(jnp.int32), pad, constant_values=_VOCAB)
    idx = idx.reshape(nblk, 1, rows)
    tgt = tgt.reshape(nblk, 1, rows)

    logits, loss_sum = pl.pallas_call(
        _fused_kernel,
        grid=(nblk,),
        in_specs=[
            pl.BlockSpec((1, 1, rows), lambda i: (i, 0, 0)),
            pl.BlockSpec((1, 1, rows), lambda i: (i, 0, 0)),
            pl.BlockSpec((_VOCAB, _VOCAB), lambda i: (0, 0)),
        ],
        out_specs=[
            pl.BlockSpec((_BB, t, _VOCAB), lambda i: (i, 0, 0)),
            pl.BlockSpec((1, 1), lambda i: (0, 0)),
        ],
        out_shape=[
            jax.ShapeDtypeStruct((b, t, _VOCAB), jnp.float32),
            jax.ShapeDtypeStruct((1, 1), jnp.float32),
        ],
        scratch_shapes=[pltpu.VMEM((_VOCAB, _VOCAB + 2), jnp.bfloat16)],
    )(idx, tgt, table)

    loss = loss_sum[0, 0] / n
    return (logits, loss)


# speedup vs baseline: 2.3636x; 2.3175x over previous
import functools

import jax
import jax.numpy as jnp
from jax.experimental import pallas as pl
from jax.experimental.pallas import tpu as pltpu

_VOCAB = 1000


def _fused_kernel(idx_ref, tgt_ref, table_ref, out_ref, loss_ref, aug_ref):
    nb = out_ref.shape[2]

    @pl.when(pl.program_id(0) == 0)
    def _prep():
        tab = table_ref[...]
        m = jnp.max(tab, axis=1, keepdims=True)
        lse = m + jnp.log(jnp.sum(jnp.exp(tab - m), axis=1, keepdims=True))
        hi = lse.astype(jnp.bfloat16)
        lo = (lse - hi.astype(jnp.float32)).astype(jnp.bfloat16)
        aug_ref[:_VOCAB, :] = tab.astype(jnp.bfloat16).T
        aug_ref[_VOCAB:_VOCAB + 1, :] = hi.T
        aug_ref[_VOCAB + 1:_VOCAB + 2, :] = lo.T
        loss_ref[...] = jnp.zeros((1, 1), jnp.float32)

    idx_row = idx_ref[0, 0, :]
    tgt_row = tgt_ref[0, 0, :]
    viota = jax.lax.broadcasted_iota(jnp.int32, (_VOCAB, nb), 0)
    onehot_t = (viota == idx_row[None, :]).astype(jnp.bfloat16)
    res = jnp.dot(aug_ref[...], onehot_t, preferred_element_type=jnp.float32)
    out_ref[0] = res[:_VOCAB, :]
    lse_sum = jnp.sum(res[_VOCAB:_VOCAB + 2, :])
    picked = jnp.sum(jnp.where(viota == tgt_row[None, :], res[:_VOCAB, :], 0.0))
    loss_ref[...] += (lse_sum - picked).reshape(1, 1)


@functools.partial(jax.jit, static_argnames=())
def kernel(table, index, targets):
    b, t = index.shape
    n = b * t
    idx = index.T.reshape(t, 1, b).astype(jnp.int32)
    tgt = targets.T.reshape(t, 1, b).astype(jnp.int32)

    logits_t, loss_sum = pl.pallas_call(
        _fused_kernel,
        grid=(t,),
        in_specs=[
            pl.BlockSpec((1, 1, b), lambda i: (i, 0, 0)),
            pl.BlockSpec((1, 1, b), lambda i: (i, 0, 0)),
            pl.BlockSpec((_VOCAB, _VOCAB), lambda i: (0, 0)),
        ],
        out_specs=[
            pl.BlockSpec((1, _VOCAB, b), lambda i: (i, 0, 0)),
            pl.BlockSpec((1, 1), lambda i: (0, 0)),
        ],
        out_shape=[
            jax.ShapeDtypeStruct((t, _VOCAB, b), jnp.float32),
            jax.ShapeDtypeStruct((1, 1), jnp.float32),
        ],
        scratch_shapes=[pltpu.VMEM((_VOCAB + 2, _VOCAB), jnp.bfloat16)],
    )(idx, tgt, table)

    logits = jnp.transpose(logits_t, (2, 0, 1))
    loss = loss_sum[0, 0] / n
    return (logits, loss)


# speedup vs baseline: 2.5654x; 1.0854x over previous
import functools

import jax
import jax.numpy as jnp
from jax.experimental import pallas as pl
from jax.experimental.pallas import tpu as pltpu
from jax.experimental.pallas import tpu_sc as plsc

_VOCAB = 1000
_SC_CORES = 2
_SC_SUBCORES = 16
_SC_LANES = 16
_SC_WIN = 256


def _tc_kernel(idx_ref, table_ref, out_ref, loss_ref, aug_ref):
    nb = out_ref.shape[2]

    @pl.when(pl.program_id(0) == 0)
    def _prep():
        tab = table_ref[...]
        m = jnp.max(tab, axis=1, keepdims=True)
        lse = m + jnp.log(jnp.sum(jnp.exp(tab - m), axis=1, keepdims=True))
        hi = lse.astype(jnp.bfloat16)
        lo = (lse - hi.astype(jnp.float32)).astype(jnp.bfloat16)
        aug_ref[:_VOCAB, :] = tab.astype(jnp.bfloat16).T
        aug_ref[_VOCAB:_VOCAB + 1, :] = hi.T
        aug_ref[_VOCAB + 1:_VOCAB + 2, :] = lo.T
        loss_ref[...] = jnp.zeros((1, 1), jnp.float32)

    idx_row = idx_ref[0, 0, :]
    viota = jax.lax.broadcasted_iota(jnp.int32, (_VOCAB, nb), 0)
    onehot_t = (viota == idx_row[None, :]).astype(jnp.bfloat16)
    res = jnp.dot(aug_ref[...], onehot_t, preferred_element_type=jnp.float32)
    out_ref[0] = res[:_VOCAB, :]
    loss_ref[...] += jnp.sum(res[_VOCAB:_VOCAB + 2, :]).reshape(1, 1)


def _sc_picked_partials(table_flat, flat_idx):
    nidx = flat_idx.shape[1]
    mesh = plsc.VectorSubcoreMesh(core_axis_name="core",
                                  subcore_axis_name="subcore")

    @pl.kernel(
        out_type=jax.ShapeDtypeStruct((_SC_CORES, _SC_SUBCORES, _SC_LANES),
                                      jnp.float32),
        mesh=mesh,
        scratch_types=[pltpu.VMEM((_SC_WIN,), jnp.float32),
                       pltpu.VMEM((_SC_LANES,), jnp.float32)],
    )
    def kern(tab_hbm, idx_hbm, o_hbm, gath_vmem, acc_vmem):
        acc_vmem[...] = jnp.zeros((_SC_LANES,), jnp.float32)

        def body(i_vmem):
            pltpu.sync_copy(tab_hbm.at[i_vmem.at[0]], gath_vmem)

            @pl.loop(0, _SC_WIN, step=_SC_LANES)
            def _(c):
                acc_vmem[...] += gath_vmem[pl.ds(c, _SC_LANES)]

        pltpu.emit_pipeline(
            body,
            grid=(nidx // _SC_WIN,),
            in_specs=[pl.BlockSpec((1, _SC_WIN), index_map=lambda i: (0, i))],
            out_specs=[],
            core_axis_name=("core", "subcore"),
            dimension_semantics=(pltpu.PARALLEL,),
        )(idx_hbm)

        core = jax.lax.axis_index("core")
        sub = jax.lax.axis_index("subcore")
        pltpu.sync_copy(acc_vmem, o_hbm.at[core, sub])

    return kern(table_flat, flat_idx)


@functools.partial(jax.jit, static_argnames=())
def kernel(table, index, targets):
    b, t = index.shape
    n = b * t
    idx = index.T.reshape(t, 1, b).astype(jnp.int32)

    flat_idx = (index.astype(jnp.int32) * _VOCAB
                + targets.astype(jnp.int32)).reshape(1, n)
    table_flat = table.reshape(_VOCAB * _VOCAB)
    picked_partials = _sc_picked_partials(table_flat, flat_idx)

    logits_t, lse_sum = pl.pallas_call(
        _tc_kernel,
        grid=(t,),
        in_specs=[
            pl.BlockSpec((1, 1, b), lambda i: (i, 0, 0)),
            pl.BlockSpec((_VOCAB, _VOCAB), lambda i: (0, 0)),
        ],
        out_specs=[
            pl.BlockSpec((1, _VOCAB, b), lambda i: (i, 0, 0)),
            pl.BlockSpec((1, 1), lambda i: (0, 0)),
        ],
        out_shape=[
            jax.ShapeDtypeStruct((t, _VOCAB, b), jnp.float32),
            jax.ShapeDtypeStruct((1, 1), jnp.float32),
        ],
        scratch_shapes=[pltpu.VMEM((_VOCAB + 2, _VOCAB), jnp.bfloat16)],
    )(idx, table)

    logits = jnp.transpose(logits_t, (2, 0, 1))
    loss = (lse_sum[0, 0] - jnp.sum(picked_partials)) / n
    return (logits, loss)
